# baseline (speedup 1.0000x reference)
import functools

import jax
import jax.numpy as jnp
from jax import lax
from jax.experimental import pallas as pl
from jax.experimental.pallas import tpu as pltpu
from jax.experimental.pallas import tpu_sc as plsc

_VOCAB = 1000000
_DIM = 32
_ROWS = 16384
_COLS = 50
_PADL = 128
_PADC = 56

_NC = 2
_NS = 16
_NW = _NC * _NS
_RPW = _ROWS // _NW
_G = 16
_GL = 56
_NCHUNKS = _RPW // _G

_mesh = plsc.VectorSubcoreMesh(core_axis_name="c", subcore_axis_name="s")

_TR = 800
_NBLK = _VOCAB // _TR
_RSTRIDE = _DIM + 1


@functools.partial(
    pl.kernel,
    out_type=jax.ShapeDtypeStruct((_VOCAB, _DIM), jnp.float32),
    mesh=_mesh,
    compiler_params=pltpu.CompilerParams(use_tc_tiling_on_sc=False,
                                         needs_layout_passes=False),
    scratch_types=[
        pltpu.VMEM((_DIM, _TR), jnp.float32),
        pltpu.VMEM((_TR, _RSTRIDE), jnp.float32),
    ],
)
def _transpose_kernel(tt_hbm, out_hbm, colbuf, rowbuf):
    wid = lax.axis_index("s") * _NC + lax.axis_index("c")
    lane16 = lax.iota(jnp.int32, 16)

    def do_block(t, _):
        b = wid + t * _NW

        @pl.when(b < _NBLK)
        def _():
            c0 = b * _TR
            pltpu.sync_copy(tt_hbm.at[pl.ds(0, _DIM), pl.ds(c0, _TR)],
                            colbuf)

            def do_k(k, _):
                ridx = lane16 + k * 16
                for d in range(_DIM):
                    v = colbuf[d, pl.ds(k * 16, 16)]
                    dv = lane16 * 0 + d
                    plsc.store_scatter(rowbuf, [ridx, dv], v)
                return 0
            lax.fori_loop(0, _TR // 16, do_k, 0)
            pltpu.sync_copy(rowbuf.at[pl.ds(0, _TR), pl.ds(0, _DIM)],
                            out_hbm.at[pl.ds(c0, _TR)])
        return 0
    lax.fori_loop(0, (_NBLK + _NW - 1) // _NW, do_block, 0)


@functools.partial(
    pl.kernel,
    out_type=jax.ShapeDtypeStruct((_ROWS, _PADC, _PADL), jnp.float32),
    mesh=_mesh,
    compiler_params=pltpu.CompilerParams(use_tc_tiling_on_sc=False),
    scratch_types=[
        pltpu.VMEM((_RPW, _PADL), jnp.int32),
        pltpu.VMEM((_G * _GL, _DIM), jnp.float32),
        pltpu.VMEM((_G * _GL, _DIM), jnp.float32),
        pltpu.SemaphoreType.DMA,
        pltpu.SemaphoreType.DMA,
        pltpu.SemaphoreType.DMA,
    ],
)
def _gather_kernel(x_hbm, table_hbm, out_hbm, xrows_v, rows0, rows1,
                   gsem, ssem0, ssem1):
    wid = lax.axis_index("s") * _NC + lax.axis_index("c")
    base = wid * _RPW
    pltpu.sync_copy(x_hbm.at[pl.ds(base, _RPW)], xrows_v)

    bufs = (rows0, rows1)
    ssems = (ssem0, ssem1)
    ghandles = [[], []]
    shandles = [[], []]

    def fire(c):
        b = c % 2
        for j in range(_G):
            r = c * _G + j
            ghandles[b].append(pltpu.async_copy(
                table_hbm.at[xrows_v.at[r, pl.ds(0, _GL)]],
                bufs[b].at[pl.ds(j * _GL, _GL)], gsem))

    fire(0)
    for c in range(_NCHUNKS):
        b = c % 2
        for h in ghandles[b]:
            h.wait()
        ghandles[b] = []
        for h in shandles[1 - b]:
            h.wait()
        shandles[1 - b] = []
        if c + 1 < _NCHUNKS:
            fire(c + 1)
        row0 = base + c * _G
        for j in range(_G):
            shandles[b].append(pltpu.async_copy(
                bufs[b].at[pl.ds(j * _GL, _COLS)],
                out_hbm.at[row0 + j, pl.ds(0, _COLS), pl.ds(0, _DIM)],
                ssems[b]))
    for h in shandles[0] + shandles[1]:
        h.wait()


def kernel(x, table):
    x128 = jnp.pad(x, ((0, 0), (0, _PADL - _COLS)), mode="edge")
    table_rm = _transpose_kernel(table.T)
    mid = _gather_kernel(x128, table_rm)
    return mid[:, :_COLS, :_DIM]


# speedup vs baseline: 4.2929x; 4.2929x over previous
import functools

import jax
import jax.numpy as jnp
from jax import lax
from jax.experimental import pallas as pl
from jax.experimental.pallas import tpu as pltpu
from jax.experimental.pallas import tpu_sc as plsc

_VOCAB = 1000000
_DIM = 32
_ROWS = 16384
_COLS = 50
_PADL = 128
_PADC = 56

_NC = 2
_NS = 16
_NW = _NC * _NS
_RPW = _ROWS // _NW
_G = 16
_GL = 56
_NCHUNKS = _RPW // _G

_mesh = plsc.VectorSubcoreMesh(core_axis_name="c", subcore_axis_name="s")


@functools.partial(
    pl.kernel,
    out_type=jax.ShapeDtypeStruct((_ROWS, _PADC, _PADL), jnp.float32),
    mesh=_mesh,
    compiler_params=pltpu.CompilerParams(use_tc_tiling_on_sc=False),
    scratch_types=[
        pltpu.VMEM((_RPW, _PADL), jnp.int32),
        pltpu.VMEM((_G * _GL, _DIM), jnp.float32),
        pltpu.VMEM((_G * _GL, _DIM), jnp.float32),
        pltpu.SemaphoreType.DMA,
        pltpu.SemaphoreType.DMA,
        pltpu.SemaphoreType.DMA,
    ],
)
def _gather_kernel(x_hbm, table_hbm, out_hbm, xrows_v, rows0, rows1,
                   gsem, ssem0, ssem1):
    wid = lax.axis_index("s") * _NC + lax.axis_index("c")
    base = wid * _RPW
    pltpu.sync_copy(x_hbm.at[pl.ds(base, _RPW)], xrows_v)

    bufs = (rows0, rows1)
    ssems = (ssem0, ssem1)
    ghandles = [[], []]
    shandles = [[], []]

    def fire(c):
        b = c % 2
        for j in range(_G):
            r = c * _G + j
            ghandles[b].append(pltpu.async_copy(
                table_hbm.at[xrows_v.at[r, pl.ds(0, _GL)]],
                bufs[b].at[pl.ds(j * _GL, _GL)], gsem))

    fire(0)
    for c in range(_NCHUNKS):
        b = c % 2
        for h in ghandles[b]:
            h.wait()
        ghandles[b] = []
        for h in shandles[1 - b]:
            h.wait()
        shandles[1 - b] = []
        if c + 1 < _NCHUNKS:
            fire(c + 1)
        row0 = base + c * _G
        for j in range(_G):
            shandles[b].append(pltpu.async_copy(
                bufs[b].at[pl.ds(j * _GL, _COLS)],
                out_hbm.at[row0 + j, pl.ds(0, _COLS), pl.ds(0, _DIM)],
                ssems[b]))
    for h in shandles[0] + shandles[1]:
        h.wait()


def kernel(x, table):
    x128 = jnp.pad(x, ((0, 0), (0, _PADL - _COLS)), mode="edge")
    mid = _gather_kernel(x128, table)
    return mid[:, :_COLS, :_DIM]


# speedup vs baseline: 4.3805x; 1.0204x over previous
import functools

import jax
import jax.numpy as jnp
from jax import lax
from jax.experimental import pallas as pl
from jax.experimental.pallas import tpu as pltpu
from jax.experimental.pallas import tpu_sc as plsc

_VOCAB = 1000000
_DIM = 32
_ROWS = 16384
_COLS = 50
_PADL = 128
_PADC = 56

_NC = 2
_NS = 16
_NW = _NC * _NS
_RPW = _ROWS // _NW
_G = 16
_GL = 56
_NCHUNKS = _RPW // _G

_mesh = plsc.VectorSubcoreMesh(core_axis_name="c", subcore_axis_name="s")


@functools.partial(
    pl.kernel,
    out_type=jax.ShapeDtypeStruct((_ROWS, _PADC, _PADL), jnp.float32),
    mesh=_mesh,
    compiler_params=pltpu.CompilerParams(use_tc_tiling_on_sc=False),
    scratch_types=[
        pltpu.VMEM((_RPW, _PADL), jnp.int32),
        pltpu.VMEM((_G * _GL, _DIM), jnp.float32),
        pltpu.VMEM((_G * _GL, _DIM), jnp.float32),
        pltpu.SemaphoreType.DMA,
        pltpu.SemaphoreType.DMA,
        pltpu.SemaphoreType.DMA,
    ],
)
def _gather_kernel(x_hbm, table_hbm, out_hbm, xrows_v, rows0, rows1,
                   gsem, ssem0, ssem1):
    wid = lax.axis_index("s") * _NC + lax.axis_index("c")
    base = wid * _RPW
    pltpu.sync_copy(x_hbm.at[pl.ds(base, _RPW)], xrows_v)

    bufs = (rows0, rows1)
    ssems = (ssem0, ssem1)
    ghandles = [[], []]
    shandles = [[], []]

    def fire(c):
        b = c % 2
        for j in range(_G):
            r = c * _G + j
            ghandles[b].append(pltpu.async_copy(
                table_hbm.at[xrows_v.at[r, pl.ds(0, _GL)]],
                bufs[b].at[pl.ds(j * _GL, _GL)], gsem))

    fire(0)
    for c in range(_NCHUNKS):
        b = c % 2
        for h in shandles[1 - b]:
            h.wait()
        shandles[1 - b] = []
        if c + 1 < _NCHUNKS:
            fire(c + 1)
        for h in ghandles[b]:
            h.wait()
        ghandles[b] = []
        row0 = base + c * _G
        for j in range(_G):
            shandles[b].append(pltpu.async_copy(
                bufs[b].at[pl.ds(j * _GL, _COLS)],
                out_hbm.at[row0 + j, pl.ds(0, _COLS), pl.ds(0, _DIM)],
                ssems[b]))
    for h in shandles[0] + shandles[1]:
        h.wait()


def kernel(x, table):
    x128 = jnp.pad(x, ((0, 0), (0, _PADL - _COLS)), mode="edge")
    mid = _gather_kernel(x128, table)
    return mid[:, :_COLS, :_DIM]
